# baseline (device time: 398273 ns/iter reference)
import jax
import jax.numpy as jnp
from jax import lax
from jax.experimental import pallas as pl
from jax.experimental.pallas import tpu as pltpu

N_DEV = 8
M = 4096
K_SHARD = 512
N_OUT = 2048
HALF = N_OUT // 2
CHUNK = M // N_DEV
SUB = CHUNK // 2

N_STEPS = 2 * (N_DEV - 1)


def _silu(v):
    return v * (1.0 / (1.0 + jnp.exp(-v)))


def _body(x_ref, w_ref, out_hbm, comm_cw, comm_ccw,
          send_cw, recv_cw, send_ccw, recv_ccw,
          out_sem_a, out_sem_b, credit_cw, credit_ccw):
    my = lax.axis_index("i")
    left = lax.rem(my + N_DEV - 1, N_DEV)
    right = lax.rem(my + 1, N_DEV)

    def pchunk(c, lo, hi):
        return jnp.dot(
            x_ref[pl.ds(c * CHUNK, CHUNK), :], w_ref[:, lo:hi],
            preferred_element_type=jnp.float32,
        )

    barrier_sem = pltpu.get_barrier_semaphore()
    for nbr in (left, right):
        pl.semaphore_signal(
            barrier_sem, inc=1,
            device_id=(nbr,), device_id_type=pl.DeviceIdType.MESH,
        )
    pl.semaphore_wait(barrier_sem, 2)

    comm_cw[0] = pchunk(my, 0, HALF)
    comm_ccw[0] = pchunk(my, HALF, N_OUT)

    pending_stores = []
    for t in range(N_STEPS):
        s_slot = t % 2
        r_slot = (t + 1) % 2
        if t >= 1:
            pl.semaphore_wait(credit_cw, 1)
            pl.semaphore_wait(credit_ccw, 1)
        rdmas = {}
        for ring, comm, ssem, rsem, dev in (
            ("cw", comm_cw, send_cw, recv_cw, right),
            ("ccw", comm_ccw, send_ccw, recv_ccw, left),
        ):
            for sub in range(2):
                r = pltpu.make_async_remote_copy(
                    src_ref=comm.at[s_slot, pl.ds(sub * SUB, SUB), :],
                    dst_ref=comm.at[r_slot, pl.ds(sub * SUB, SUB), :],
                    send_sem=ssem.at[s_slot, sub],
                    recv_sem=rsem.at[r_slot, sub],
                    device_id=(dev,), device_id_type=pl.DeviceIdType.MESH,
                )
                r.start()
                rdmas[(ring, sub)] = r
        if t < N_DEV - 1:
            c_cw = lax.rem(my - (t + 1) + N_DEV, N_DEV)
            c_ccw = lax.rem(my + t + 1, N_DEV)
            pa = pchunk(c_cw, 0, HALF)
            pb = pchunk(c_ccw, HALF, N_OUT)

        for sub in range(2):
            lo, hi = sub * SUB, (sub + 1) * SUB
            for ring, comm, p in (("cw", comm_cw, "pa"), ("ccw", comm_ccw, "pb")):
                rdmas[(ring, sub)].wait()
                v = pa if p == "pa" else pb
                if t < N_DEV - 2:
                    comm[r_slot, lo:hi, :] = comm[r_slot, lo:hi, :] + v[lo:hi, :]
                elif t == N_DEV - 2:
                    comm[r_slot, lo:hi, :] = _silu(
                        comm[r_slot, lo:hi, :] + v[lo:hi, :])

        if pending_stores:
            for st in pending_stores:
                st.wait()
            pending_stores = []
        if t < N_STEPS - 1:
            pl.semaphore_signal(
                credit_cw, inc=1,
                device_id=(left,), device_id_type=pl.DeviceIdType.MESH,
            )
            pl.semaphore_signal(
                credit_ccw, inc=1,
                device_id=(right,), device_id_type=pl.DeviceIdType.MESH,
            )
        if t >= N_DEV - 2:
            c_a = lax.rem(my - (t - (N_DEV - 1)) + N_DEV, N_DEV)
            c_b = lax.rem(my + (t - (N_DEV - 1)) + N_DEV, N_DEV)
            st_a = pltpu.make_async_copy(
                comm_cw.at[r_slot],
                out_hbm.at[pl.ds(c_a * CHUNK, CHUNK), pl.ds(0, HALF)],
                out_sem_a)
            st_b = pltpu.make_async_copy(
                comm_ccw.at[r_slot],
                out_hbm.at[pl.ds(c_b * CHUNK, CHUNK), pl.ds(HALF, HALF)],
                out_sem_b)
            st_a.start()
            st_b.start()
            pending_stores = [st_a, st_b]

    for st in pending_stores:
        st.wait()


def kernel(x, w_mat):
    return pl.pallas_call(
        _body,
        out_shape=jax.ShapeDtypeStruct((M, N_OUT), jnp.float32),
        in_specs=[
            pl.BlockSpec(memory_space=pltpu.VMEM),
            pl.BlockSpec(memory_space=pltpu.VMEM),
        ],
        out_specs=pl.BlockSpec(memory_space=pltpu.MemorySpace.HBM),
        scratch_shapes=[
            pltpu.VMEM((2, CHUNK, HALF), jnp.float32),
            pltpu.VMEM((2, CHUNK, HALF), jnp.float32),
            pltpu.SemaphoreType.DMA((2, 2)),
            pltpu.SemaphoreType.DMA((2, 2)),
            pltpu.SemaphoreType.DMA((2, 2)),
            pltpu.SemaphoreType.DMA((2, 2)),
            pltpu.SemaphoreType.DMA,
            pltpu.SemaphoreType.DMA,
            pltpu.SemaphoreType.REGULAR,
            pltpu.SemaphoreType.REGULAR,
        ],
        compiler_params=pltpu.CompilerParams(collective_id=0),
    )(x, w_mat)


# device time: 382530 ns/iter; 1.0412x vs baseline; 1.0412x over previous
import jax
import jax.numpy as jnp
from jax import lax
from jax.experimental import pallas as pl
from jax.experimental.pallas import tpu as pltpu

N_DEV = 8
M = 4096
K_SHARD = 512
N_OUT = 2048
HALF = N_OUT // 2
CHUNK = M // N_DEV
SUB = CHUNK // 2

N_STEPS = 2 * (N_DEV - 1)
SLOTS = 4


def _silu(v):
    return v * (1.0 / (1.0 + jnp.exp(-v)))


def _body(x_ref, w_ref, out_hbm, comm_cw, comm_ccw,
          send_cw, recv_cw, send_ccw, recv_ccw,
          out_sem_a, out_sem_b, credit_cw, credit_ccw):
    my = lax.axis_index("i")
    left = lax.rem(my + N_DEV - 1, N_DEV)
    right = lax.rem(my + 1, N_DEV)

    def pchunk(c, lo, hi):
        return jnp.dot(
            x_ref[pl.ds(c * CHUNK, CHUNK), :], w_ref[:, lo:hi],
            preferred_element_type=jnp.float32,
        )

    barrier_sem = pltpu.get_barrier_semaphore()
    for nbr in (left, right):
        pl.semaphore_signal(
            barrier_sem, inc=1,
            device_id=(nbr,), device_id_type=pl.DeviceIdType.MESH,
        )
    pl.semaphore_wait(barrier_sem, 2)

    comm_cw[0] = pchunk(my, 0, HALF)
    comm_ccw[0] = pchunk(my, HALF, N_OUT)

    pending_stores = []
    for t in range(N_STEPS):
        s_slot = t % SLOTS
        r_slot = (t + 1) % SLOTS
        if t >= SLOTS - 1:
            pl.semaphore_wait(credit_cw, 1)
            pl.semaphore_wait(credit_ccw, 1)
        rdmas = {}
        for ring, comm, ssem, rsem, dev in (
            ("cw", comm_cw, send_cw, recv_cw, right),
            ("ccw", comm_ccw, send_ccw, recv_ccw, left),
        ):
            for sub in range(2):
                r = pltpu.make_async_remote_copy(
                    src_ref=comm.at[s_slot, pl.ds(sub * SUB, SUB), :],
                    dst_ref=comm.at[r_slot, pl.ds(sub * SUB, SUB), :],
                    send_sem=ssem.at[s_slot, sub],
                    recv_sem=rsem.at[r_slot, sub],
                    device_id=(dev,), device_id_type=pl.DeviceIdType.MESH,
                )
                r.start()
                rdmas[(ring, sub)] = r
        if t < N_DEV - 1:
            c_cw = lax.rem(my - (t + 1) + N_DEV, N_DEV)
            c_ccw = lax.rem(my + t + 1, N_DEV)
            pa = pchunk(c_cw, 0, HALF)
            pb = pchunk(c_ccw, HALF, N_OUT)

        for sub in range(2):
            lo, hi = sub * SUB, (sub + 1) * SUB
            for ring, comm, p in (("cw", comm_cw, "pa"), ("ccw", comm_ccw, "pb")):
                rdmas[(ring, sub)].wait()
                v = pa if p == "pa" else pb
                if t < N_DEV - 2:
                    comm[r_slot, lo:hi, :] = comm[r_slot, lo:hi, :] + v[lo:hi, :]
                elif t == N_DEV - 2:
                    comm[r_slot, lo:hi, :] = _silu(
                        comm[r_slot, lo:hi, :] + v[lo:hi, :])

        if pending_stores:
            for st in pending_stores:
                st.wait()
            pending_stores = []
        if t < N_STEPS - (SLOTS - 1):
            pl.semaphore_signal(
                credit_cw, inc=1,
                device_id=(left,), device_id_type=pl.DeviceIdType.MESH,
            )
            pl.semaphore_signal(
                credit_ccw, inc=1,
                device_id=(right,), device_id_type=pl.DeviceIdType.MESH,
            )
        if t >= N_DEV - 2:
            c_a = lax.rem(my - (t - (N_DEV - 1)) + N_DEV, N_DEV)
            c_b = lax.rem(my + (t - (N_DEV - 1)) + N_DEV, N_DEV)
            st_a = pltpu.make_async_copy(
                comm_cw.at[r_slot],
                out_hbm.at[pl.ds(c_a * CHUNK, CHUNK), pl.ds(0, HALF)],
                out_sem_a)
            st_b = pltpu.make_async_copy(
                comm_ccw.at[r_slot],
                out_hbm.at[pl.ds(c_b * CHUNK, CHUNK), pl.ds(HALF, HALF)],
                out_sem_b)
            st_a.start()
            st_b.start()
            pending_stores = [st_a, st_b]

    for st in pending_stores:
        st.wait()


def kernel(x, w_mat):
    return pl.pallas_call(
        _body,
        out_shape=jax.ShapeDtypeStruct((M, N_OUT), jnp.float32),
        in_specs=[
            pl.BlockSpec(memory_space=pltpu.VMEM),
            pl.BlockSpec(memory_space=pltpu.VMEM),
        ],
        out_specs=pl.BlockSpec(memory_space=pltpu.MemorySpace.HBM),
        scratch_shapes=[
            pltpu.VMEM((SLOTS, CHUNK, HALF), jnp.float32),
            pltpu.VMEM((SLOTS, CHUNK, HALF), jnp.float32),
            pltpu.SemaphoreType.DMA((SLOTS, 2)),
            pltpu.SemaphoreType.DMA((SLOTS, 2)),
            pltpu.SemaphoreType.DMA((SLOTS, 2)),
            pltpu.SemaphoreType.DMA((SLOTS, 2)),
            pltpu.SemaphoreType.DMA,
            pltpu.SemaphoreType.DMA,
            pltpu.SemaphoreType.REGULAR,
            pltpu.SemaphoreType.REGULAR,
        ],
        compiler_params=pltpu.CompilerParams(collective_id=0),
    )(x, w_mat)
